# baseline (speedup 1.0000x reference)
import functools

import jax
import jax.numpy as jnp
from jax import lax
from jax.experimental import pallas as pl
from jax.experimental.pallas import tpu as pltpu
from jax.experimental.pallas import tpu_sc as plsc

ROWS = 16384
K = 512
M = 128
L = 16

_info = plsc.get_sparse_core_info()
NC = _info.num_cores
NS = _info.num_subcores
NW = NC * NS
ROWS_PER_W = ROWS // NW
R = 64
NCHUNK = ROWS_PER_W // R
NBUF = 3
GROUPS = M // L


def _sc_body(z_hbm, mask_hbm, out_hbm, mask_v, zbufs, obufs, sins, souts):
    c = lax.axis_index("c")
    s = lax.axis_index("s")
    wid = s * NC + c
    row0 = wid * ROWS_PER_W

    pltpu.sync_copy(mask_hbm, mask_v)
    mask_vecs = [mask_v[pl.ds(L * j, L)] for j in range(GROUPS)]

    def start_in(g):
        return pltpu.async_copy(z_hbm.at[pl.ds(row0 + g * R, R)],
                                zbufs[g % NBUF], sins[g % NBUF])

    in_h = [None] * NCHUNK
    out_h = [None] * NCHUNK
    for g in range(NBUF):
        in_h[g] = start_in(g)
    for g in range(NCHUNK):
        in_h[g].wait()
        if g >= NBUF:
            out_h[g - NBUF].wait()
        zb = zbufs[g % NBUF]
        ob = obufs[g % NBUF]

        @plsc.parallel_loop(0, R, unroll=8)
        def _(r):
            row_vec = jnp.full((L,), r, jnp.int32)
            for j in range(GROUPS):
                vals = plsc.load_gather(zb, [row_vec, mask_vecs[j]])
                ob[r, pl.ds(L * j, L)] = vals

        out_h[g] = pltpu.async_copy(
            ob, out_hbm.at[pl.ds(row0 + g * R, R)], souts[g % NBUF])
        if g + NBUF < NCHUNK:
            in_h[g + NBUF] = start_in(g + NBUF)

    for g in range(NCHUNK - NBUF, NCHUNK):
        out_h[g].wait()


@jax.jit
def _sc_gather(z, mask):
    mesh = plsc.VectorSubcoreMesh(core_axis_name="c", subcore_axis_name="s")
    kern = functools.partial(
        pl.kernel,
        mesh=mesh,
        compiler_params=pltpu.CompilerParams(needs_layout_passes=False),
        out_type=jax.ShapeDtypeStruct((ROWS, M), jnp.float32),
        scratch_types=[
            pltpu.VMEM((M,), jnp.int32),
            [pltpu.VMEM((R, K), jnp.float32) for _ in range(NBUF)],
            [pltpu.VMEM((R, M), jnp.float32) for _ in range(NBUF)],
            [pltpu.SemaphoreType.DMA for _ in range(NBUF)],
            [pltpu.SemaphoreType.DMA for _ in range(NBUF)],
        ],
    )(_sc_body)
    return kern(z, mask)


def kernel(z, mask):
    return _sc_gather(z, mask.astype(jnp.int32))


# speedup vs baseline: 1.0603x; 1.0603x over previous
import functools

import jax
import jax.numpy as jnp
from jax import lax
from jax.experimental import pallas as pl
from jax.experimental.pallas import tpu as pltpu
from jax.experimental.pallas import tpu_sc as plsc

ROWS = 16384
K = 512
M = 128
L = 16

_info = plsc.get_sparse_core_info()
NC = _info.num_cores
NS = _info.num_subcores
NW = NC * NS
ROWS_PER_W = ROWS // NW
R = 64
NCHUNK = ROWS_PER_W // R
NBUF = 2
GROUPS = M // L


def _sc_body(z_hbm, mask_hbm, out_hbm, mask_v, zbufs, obufs, sins, souts):
    c = lax.axis_index("c")
    s = lax.axis_index("s")
    wid = s * NC + c
    row0 = wid * ROWS_PER_W

    pltpu.sync_copy(mask_hbm, mask_v)
    mask_vecs = [mask_v[pl.ds(L * j, L)] for j in range(GROUPS)]

    def start_in(g):
        return pltpu.async_copy(z_hbm.at[pl.ds(row0 + g * R, R)],
                                zbufs[g % NBUF], sins[g % NBUF])

    in_h = [None] * NCHUNK
    out_h = [None] * NCHUNK
    for g in range(NBUF):
        in_h[g] = start_in(g)
    for g in range(NCHUNK):
        in_h[g].wait()
        if g >= NBUF:
            out_h[g - NBUF].wait()
        zb = zbufs[g % NBUF]
        ob = obufs[g % NBUF]

        @plsc.parallel_loop(0, R, unroll=4)
        def _(r):
            row_vec = jnp.full((L,), r, jnp.int32)
            for j in range(GROUPS):
                vals = plsc.load_gather(zb, [row_vec, mask_vecs[j]])
                ob[r, pl.ds(L * j, L)] = vals

        out_h[g] = pltpu.async_copy(
            ob, out_hbm.at[pl.ds(row0 + g * R, R)], souts[g % NBUF])
        if g + NBUF < NCHUNK:
            in_h[g + NBUF] = start_in(g + NBUF)

    for g in range(NCHUNK - NBUF, NCHUNK):
        out_h[g].wait()


@jax.jit
def _sc_gather(z, mask):
    mesh = plsc.VectorSubcoreMesh(core_axis_name="c", subcore_axis_name="s")
    kern = functools.partial(
        pl.kernel,
        mesh=mesh,
        compiler_params=pltpu.CompilerParams(needs_layout_passes=False),
        out_type=jax.ShapeDtypeStruct((ROWS, M), jnp.float32),
        scratch_types=[
            pltpu.VMEM((M,), jnp.int32),
            [pltpu.VMEM((R, K), jnp.float32) for _ in range(NBUF)],
            [pltpu.VMEM((R, M), jnp.float32) for _ in range(NBUF)],
            [pltpu.SemaphoreType.DMA for _ in range(NBUF)],
            [pltpu.SemaphoreType.DMA for _ in range(NBUF)],
        ],
    )(_sc_body)
    return kern(z, mask)


def kernel(z, mask):
    return _sc_gather(z, mask.astype(jnp.int32))


# speedup vs baseline: 1.0818x; 1.0203x over previous
import functools

import jax
import jax.numpy as jnp
from jax import lax
from jax.experimental import pallas as pl
from jax.experimental.pallas import tpu as pltpu
from jax.experimental.pallas import tpu_sc as plsc

ROWS = 16384
K = 512
M = 128
L = 16

_info = plsc.get_sparse_core_info()
NC = _info.num_cores
NS = _info.num_subcores
NW = NC * NS
ROWS_PER_W = ROWS // NW
R = 64
NCHUNK = ROWS_PER_W // R
NBUF = 3
GROUPS = M // L


def _sc_body(z_hbm, mask_hbm, out_hbm, mask_v, zbufs, obufs, sins, souts):
    c = lax.axis_index("c")
    s = lax.axis_index("s")
    wid = s * NC + c
    row0 = wid * ROWS_PER_W

    pltpu.sync_copy(mask_hbm, mask_v)
    mask_vecs = [mask_v[pl.ds(L * j, L)] for j in range(GROUPS)]

    def start_in(g):
        return pltpu.async_copy(z_hbm.at[pl.ds(row0 + g * R, R)],
                                zbufs[g % NBUF], sins[g % NBUF])

    in_h = [None] * NCHUNK
    out_h = [None] * NCHUNK
    for g in range(NBUF):
        in_h[g] = start_in(g)
    for g in range(NCHUNK):
        in_h[g].wait()
        if g >= NBUF:
            out_h[g - NBUF].wait()
        zb = zbufs[g % NBUF]
        ob = obufs[g % NBUF]

        @plsc.parallel_loop(0, R, unroll=2)
        def _(r):
            row_vec = jnp.full((L,), r, jnp.int32)
            for j in range(GROUPS):
                vals = plsc.load_gather(zb, [row_vec, mask_vecs[j]])
                ob[r, pl.ds(L * j, L)] = vals

        out_h[g] = pltpu.async_copy(
            ob, out_hbm.at[pl.ds(row0 + g * R, R)], souts[g % NBUF])
        if g + NBUF < NCHUNK:
            in_h[g + NBUF] = start_in(g + NBUF)

    for g in range(NCHUNK - NBUF, NCHUNK):
        out_h[g].wait()


@jax.jit
def _sc_gather(z, mask):
    mesh = plsc.VectorSubcoreMesh(core_axis_name="c", subcore_axis_name="s")
    kern = functools.partial(
        pl.kernel,
        mesh=mesh,
        compiler_params=pltpu.CompilerParams(needs_layout_passes=False),
        out_type=jax.ShapeDtypeStruct((ROWS, M), jnp.float32),
        scratch_types=[
            pltpu.VMEM((M,), jnp.int32),
            [pltpu.VMEM((R, K), jnp.float32) for _ in range(NBUF)],
            [pltpu.VMEM((R, M), jnp.float32) for _ in range(NBUF)],
            [pltpu.SemaphoreType.DMA for _ in range(NBUF)],
            [pltpu.SemaphoreType.DMA for _ in range(NBUF)],
        ],
    )(_sc_body)
    return kern(z, mask)


def kernel(z, mask):
    return _sc_gather(z, mask.astype(jnp.int32))


# speedup vs baseline: 1.1154x; 1.0311x over previous
import functools

import jax
import jax.numpy as jnp
from jax import lax
from jax.experimental import pallas as pl
from jax.experimental.pallas import tpu as pltpu
from jax.experimental.pallas import tpu_sc as plsc

ROWS = 16384
K = 512
M = 128
L = 16

_info = plsc.get_sparse_core_info()
NC = _info.num_cores
NS = _info.num_subcores
NW = NC * NS
ROWS_PER_W = ROWS // NW
R = 64
NCHUNK = ROWS_PER_W // R
NBUF = 3
GROUPS = M // L


def _sc_body(z_hbm, mask_hbm, out_hbm, mask_v, zbufs, obufs, sins, souts):
    c = lax.axis_index("c")
    s = lax.axis_index("s")
    wid = s * NC + c
    row0 = wid * ROWS_PER_W

    def start_in(g):
        return pltpu.async_copy(z_hbm.at[pl.ds(row0 + g * R, R)],
                                zbufs[g % NBUF], sins[g % NBUF])

    in_h = [None] * NCHUNK
    out_h = [None] * NCHUNK
    for g in range(NBUF):
        in_h[g] = start_in(g)

    pltpu.sync_copy(mask_hbm, mask_v)
    mask_vecs = [mask_v[pl.ds(L * j, L)] for j in range(GROUPS)]
    for g in range(NCHUNK):
        in_h[g].wait()
        if g >= NBUF:
            out_h[g - NBUF].wait()
        zb = zbufs[g % NBUF]
        ob = obufs[g % NBUF]

        @plsc.parallel_loop(0, R, unroll=2)
        def _(r):
            row_vec = jnp.full((L,), r, jnp.int32)
            for j in range(GROUPS):
                vals = plsc.load_gather(zb, [row_vec, mask_vecs[j]])
                ob[r, pl.ds(L * j, L)] = vals

        out_h[g] = pltpu.async_copy(
            ob, out_hbm.at[pl.ds(row0 + g * R, R)], souts[g % NBUF])
        if g + NBUF < NCHUNK:
            in_h[g + NBUF] = start_in(g + NBUF)

    for g in range(NCHUNK - NBUF, NCHUNK):
        out_h[g].wait()


@jax.jit
def _sc_gather(z, mask):
    mesh = plsc.VectorSubcoreMesh(core_axis_name="c", subcore_axis_name="s")
    kern = functools.partial(
        pl.kernel,
        mesh=mesh,
        compiler_params=pltpu.CompilerParams(needs_layout_passes=False),
        out_type=jax.ShapeDtypeStruct((ROWS, M), jnp.float32),
        scratch_types=[
            pltpu.VMEM((M,), jnp.int32),
            [pltpu.VMEM((R, K), jnp.float32) for _ in range(NBUF)],
            [pltpu.VMEM((R, M), jnp.float32) for _ in range(NBUF)],
            [pltpu.SemaphoreType.DMA for _ in range(NBUF)],
            [pltpu.SemaphoreType.DMA for _ in range(NBUF)],
        ],
    )(_sc_body)
    return kern(z, mask)


def kernel(z, mask):
    return _sc_gather(z, mask.astype(jnp.int32))
